# baseline (speedup 1.0000x reference)
import functools

import numpy as np
import jax
import jax.numpy as jnp
from jax import lax
from jax.experimental import pallas as pl
from jax.experimental.pallas import tpu as pltpu
from jax.experimental.pallas import tpu_sc as plsc

N = 10000
E = 320000
HEADS = 4
LANES = 16
NC, NS = 2, 16
NW = NC * NS
EPT = E // NW
NPAD = 10240
ROWS_PER = NPAD // NS
BN = 1000


def _make_edge_kernel(D, CK, dbuf, nheads=HEADS):
    ROW = D + 16
    NCH = EPT // CK
    G = D // LANES
    HD = D // nheads
    NB = 2 if dbuf else 1
    mesh = plsc.VectorSubcoreMesh(core_axis_name="c", subcore_axis_name="s")

    scratch = [pltpu.VMEM_SHARED((NPAD, ROW), jnp.float32)]
    for _ in range(NB):
        scratch += [pltpu.VMEM((CK,), jnp.int32),
                    pltpu.VMEM((CK,), jnp.int32),
                    pltpu.VMEM((CK, ROW), jnp.float32),
                    pltpu.VMEM((CK, 16), jnp.float32),
                    pltpu.SemaphoreType.DMA,
                    pltpu.SemaphoreType.DMA,
                    pltpu.SemaphoreType.DMA,
                    pltpu.SemaphoreType.DMA]
    scratch += [pltpu.VMEM((CK, ROW), jnp.float32)]

    @functools.partial(
        pl.kernel,
        out_type=jax.ShapeDtypeStruct((NC, NPAD, ROW), jnp.float32),
        mesh=mesh,
        compiler_params=pltpu.CompilerParams(use_tc_tiling_on_sc=False),
        scratch_types=scratch,
    )
    def edge_kernel(featx, qd, src, dst, zeros, out, accum, *bufs):
        c = lax.axis_index("c")
        s = lax.axis_index("s")
        wid = s * NC + c
        r0 = s * ROWS_PER
        sets = [bufs[8 * i:8 * i + 8] for i in range(NB)]
        msg_buf = bufs[8 * NB]
        pltpu.sync_copy(zeros.at[pl.ds(r0, ROWS_PER)],
                        accum.at[pl.ds(r0, ROWS_PER)])
        plsc.subcore_barrier()
        ebase = wid * EPT
        lane = lax.iota(jnp.int32, 16)

        def prefetch(ci, p):
            sb, db, fb, qb, sf, sq, si, sj = sets[p]
            base = ebase + ci * CK
            ci1 = pltpu.async_copy(src.at[pl.ds(base, CK)], sb, si)
            ci2 = pltpu.async_copy(dst.at[pl.ds(base, CK)], db, sj)
            ci1.wait()
            ci2.wait()
            pltpu.async_copy(featx.at[sb], fb, sf)
            pltpu.async_copy(qd.at[db], qb, sq)

        def process(p):
            sb, db, fb, qb, sf, sq, si, sj = sets[p]
            pltpu.make_async_copy(featx.at[sb], fb, sf).wait()
            pltpu.make_async_copy(qd.at[db], qb, sq).wait()

            def edge_body(k, carry2):
                ev = fb[k, pl.ds(D, 16)] + qb[k, :]
                e = jnp.where(ev >= 0.0, ev, ev * 0.2)
                w = jnp.exp(e)
                msg_buf[k, pl.ds(D, 16)] = w
                for g in range(G):
                    ha = (16 * g) // HD
                    hb = (16 * g + 15) // HD
                    if ha == hb:
                        wb = w[ha]
                    else:
                        wb = jnp.where(lane < hb * HD - 16 * g, w[ha], w[hb])
                    msg_buf[k, pl.ds(16 * g, 16)] = (
                        fb[k, pl.ds(16 * g, 16)] * wb)
                return carry2

            lax.fori_loop(0, CK, edge_body, 0, unroll=4)
            pltpu.sync_copy(msg_buf, accum.at[db], add=True)

        if dbuf:
            prefetch(0, 0)

            def pair_body(i, carry):
                prefetch(2 * i + 1, 1)
                process(0)

                @pl.when(2 * i + 2 < NCH)
                def _():
                    prefetch(2 * i + 2, 0)

                process(1)
                return carry

            lax.fori_loop(0, NCH // 2, pair_body, 0)
            if NCH % 2 == 1:
                process(0)
        else:
            def chunk_body(ci, carry):
                prefetch(ci, 0)
                process(0)
                return carry

            lax.fori_loop(0, NCH, chunk_body, 0)
        plsc.subcore_barrier()
        pltpu.sync_copy(accum.at[pl.ds(r0, ROWS_PER)],
                        out.at[c, pl.ds(r0, ROWS_PER)])

    return edge_kernel


_EDGE_K = {128: _make_edge_kernel(128, 80, True),
           80: _make_edge_kernel(80, 80, True, nheads=2)}


def _mm_body(x_ref, w_ref, o_ref):
    o_ref[...] = jnp.dot(x_ref[...], w_ref[...],
                         preferred_element_type=jnp.float32)


def _matmul(x, w):
    k = w.shape[1]
    return pl.pallas_call(
        _mm_body,
        grid=(N // BN,),
        in_specs=[pl.BlockSpec((BN, x.shape[1]), lambda i: (i, 0)),
                  pl.BlockSpec((x.shape[1], k), lambda i: (0, 0))],
        out_specs=pl.BlockSpec((BN, k), lambda i: (i, 0)),
        out_shape=jax.ShapeDtypeStruct((N, k), jnp.float32),
    )(x, w)


def _post01_body(p0_ref, p1_ref, sk_ref, sb_ref, b_ref, o_ref):
    lane = lax.broadcasted_iota(jnp.int32, (BN, 256), 1)
    p = jnp.where(lane < 132, p0_ref[...] + p1_ref[...], 0.0)
    denb = jnp.dot(p, b_ref[...], preferred_element_type=jnp.float32) + 1e-9
    msg = jnp.where(lane < 128, p / denb, 0.0)
    r = msg + sk_ref[...] + sb_ref[...]
    o_ref[...] = jnp.where(r > 0.0, r, jnp.exp(r) - 1.0)


def _post2_body(p0_ref, p1_ref, sk_ref, sb_ref, b_ref, s_ref, o_ref):
    lane = lax.broadcasted_iota(jnp.int32, (BN, 256), 1)
    p = jnp.where(lane < 164, p0_ref[...] + p1_ref[...], 0.0)
    denb = jnp.dot(p, b_ref[...], preferred_element_type=jnp.float32) + 1e-9
    msg = jnp.where(lane < 160, p / denb, 0.0)
    ms = jnp.dot(msg, s_ref[...], preferred_element_type=jnp.float32)
    m = 0.25 * ms + sk_ref[...] + sb_ref[...]
    mm = jnp.where(lane < 40, m, -1e30)
    mx = jnp.max(mm, axis=1, keepdims=True)
    ex = jnp.where(lane < 40, jnp.exp(mm - mx), 0.0)
    lse = jnp.log(jnp.sum(ex, axis=1, keepdims=True))
    o_ref[...] = mm - mx - lse


def _full_spec(shape):
    return pl.BlockSpec(shape, lambda i: tuple(0 for _ in shape))


def _post_call(body, args, extra_consts):
    specs = [pl.BlockSpec((BN, 256), lambda i: (i, 0)),
             pl.BlockSpec((BN, 256), lambda i: (i, 0)),
             pl.BlockSpec((BN, 256), lambda i: (i, 0)),
             _full_spec((1, 256))]
    specs += [_full_spec((256, 256)) for _ in extra_consts]
    return pl.pallas_call(
        body,
        grid=(N // BN,),
        in_specs=specs,
        out_specs=pl.BlockSpec((BN, 256), lambda i: (i, 0)),
        out_shape=jax.ShapeDtypeStruct((N, 256), jnp.float32),
    )(*args, *extra_consts)


def _np_bmat(d, hd):
    b = np.zeros((256, 256), np.float32)
    for h in range(HEADS):
        b[d + h, hd * h:hd * h + hd] = 1.0
    return jnp.asarray(b)


def _np_smat(hd):
    s = np.zeros((256, 256), np.float32)
    for h in range(HEADS):
        for j in range(hd):
            s[hd * h + j, j] = 1.0
    return jnp.asarray(s)


def _pad_cols(a, w):
    return jnp.pad(a, ((0, 0), (0, w - a.shape[1])))


def _layer_pre(h, W, al, ar, sW, D):
    hd = D // HEADS
    w3 = W.reshape(W.shape[0], HEADS, hd)
    wal = jnp.sum(w3 * al[None], axis=-1)
    war = jnp.sum(w3 * ar[None], axis=-1)
    wcat = jnp.concatenate([W, wal, war, sW], axis=1)
    kpad = -(-wcat.shape[1] // 128) * 128
    P = _matmul(h, _pad_cols(wcat, kpad))
    featx = _pad_cols(P[:, :D + 4], D + 16)
    qd = _pad_cols(P[:, D + 4:D + 8], 16)
    skip = P[:, D + 8:D + 8 + sW.shape[1]]
    return featx, qd, skip


def kernel(x, edge_index, W0, al0, ar0, sW0, sb0, W1, al1, ar1, sW1, sb1,
           W2, al2, ar2, sW2, sb2):
    src = edge_index[0]
    dst = edge_index[1]
    zeros01 = jnp.zeros((NPAD, 144), jnp.float32)
    zeros80 = jnp.zeros((NPAD, 96), jnp.float32)
    D2 = 160
    b01 = _np_bmat(128, 32)
    b2 = _np_bmat(160, 40)
    s2 = _np_smat(40)

    featx, qd, skip = _layer_pre(x, W0, al0, ar0, sW0, 128)
    parts = _EDGE_K[128](featx, qd, src, dst, zeros01)[:, :N]
    h = _post_call(_post01_body,
                   (_pad_cols(parts[0], 256), _pad_cols(parts[1], 256),
                    _pad_cols(skip, 256),
                    _pad_cols((sb0)[None, :], 256)),
                   (b01,))[:, :128]

    featx, qd, skip = _layer_pre(h, W1, al1, ar1, sW1, 128)
    parts = _EDGE_K[128](featx, qd, src, dst, zeros01)[:, :N]
    h = _post_call(_post01_body,
                   (_pad_cols(parts[0], 256), _pad_cols(parts[1], 256),
                    _pad_cols(skip, 256),
                    _pad_cols((sb1)[None, :], 256)),
                   (b01,))[:, :128]

    hd2 = D2 // HEADS
    w3 = W2.reshape(W2.shape[0], HEADS, hd2)
    wal = jnp.sum(w3 * al2[None], axis=-1)
    war = jnp.sum(w3 * ar2[None], axis=-1)
    wcat = jnp.concatenate([W2, wal, war, sW2], axis=1)
    P = _matmul(h, _pad_cols(wcat, 256))
    fxA = _pad_cols(jnp.concatenate([P[:, 0:80], P[:, 160:162]], axis=1), 96)
    fxB = _pad_cols(jnp.concatenate([P[:, 80:160], P[:, 162:164]], axis=1), 96)
    qdA = _pad_cols(P[:, 164:166], 16)
    qdB = _pad_cols(P[:, 166:168], 16)
    skip = P[:, 168:208]
    pA = _EDGE_K[80](fxA, qdA, src, dst, zeros80)[:, :N]
    pB = _EDGE_K[80](fxB, qdB, src, dst, zeros80)[:, :N]
    p0 = jnp.concatenate([pA[0, :, 0:80], pB[0, :, 0:80],
                          pA[0, :, 80:82], pB[0, :, 80:82]], axis=1)
    p1 = jnp.concatenate([pA[1, :, 0:80], pB[1, :, 0:80],
                          pA[1, :, 80:82], pB[1, :, 80:82]], axis=1)
    out = _post_call(_post2_body,
                     (_pad_cols(p0, 256), _pad_cols(p1, 256),
                      _pad_cols(skip, 256),
                      _pad_cols((sb2)[None, :], 256)),
                     (b2, s2))[:, :40]
    return out


# speedup vs baseline: 1.6690x; 1.6690x over previous
import functools

import numpy as np
import jax
import jax.numpy as jnp
from jax import lax
from jax.experimental import pallas as pl
from jax.experimental.pallas import tpu as pltpu
from jax.experimental.pallas import tpu_sc as plsc

N = 10000
E = 320000
HEADS = 4
LANES = 16
NC, NS = 2, 16
NW = NC * NS
EPT = E // NW
NPAD = 10240
ROWS_PER = NPAD // NS
BN = 1000


def _make_edge_kernel(D, CK, dbuf, nheads=HEADS):
    ROW = D + 16
    NCH = EPT // CK
    G = D // LANES
    HD = D // nheads
    NB = 2 if dbuf else 1
    mesh = plsc.VectorSubcoreMesh(core_axis_name="c", subcore_axis_name="s")

    scratch = [pltpu.VMEM_SHARED((NPAD, ROW), jnp.float32)]
    for _ in range(NB):
        scratch += [pltpu.VMEM((CK,), jnp.int32),
                    pltpu.VMEM((CK,), jnp.int32),
                    pltpu.VMEM((CK, ROW), jnp.float32),
                    pltpu.VMEM((CK, 16), jnp.float32),
                    pltpu.SemaphoreType.DMA,
                    pltpu.SemaphoreType.DMA,
                    pltpu.SemaphoreType.DMA,
                    pltpu.SemaphoreType.DMA]
    scratch += [pltpu.VMEM((CK, ROW), jnp.float32)]

    @functools.partial(
        pl.kernel,
        out_type=jax.ShapeDtypeStruct((NC, NPAD, ROW), jnp.float32),
        mesh=mesh,
        compiler_params=pltpu.CompilerParams(use_tc_tiling_on_sc=False),
        scratch_types=scratch,
    )
    def edge_kernel(featx, qd, src, dst, zeros, out, accum, *bufs):
        c = lax.axis_index("c")
        s = lax.axis_index("s")
        wid = s * NC + c
        r0 = s * ROWS_PER
        sets = [bufs[8 * i:8 * i + 8] for i in range(NB)]
        msg_buf = bufs[8 * NB]
        pltpu.sync_copy(zeros.at[pl.ds(r0, ROWS_PER)],
                        accum.at[pl.ds(r0, ROWS_PER)])
        plsc.subcore_barrier()
        ebase = wid * EPT
        lane = lax.iota(jnp.int32, 16)

        def prefetch(ci, p):
            sb, db, fb, qb, sf, sq, si, sj = sets[p]
            base = ebase + ci * CK
            ci1 = pltpu.async_copy(src.at[pl.ds(base, CK)], sb, si)
            ci2 = pltpu.async_copy(dst.at[pl.ds(base, CK)], db, sj)
            ci1.wait()
            ci2.wait()
            pltpu.async_copy(featx.at[sb], fb, sf)
            pltpu.async_copy(qd.at[db], qb, sq)

        def process(p):
            sb, db, fb, qb, sf, sq, si, sj = sets[p]
            pltpu.make_async_copy(featx.at[sb], fb, sf).wait()
            pltpu.make_async_copy(qd.at[db], qb, sq).wait()

            def edge_body(k, carry2):
                ev = fb[k, pl.ds(D, 16)] + qb[k, :]
                e = jnp.where(ev >= 0.0, ev, ev * 0.2)
                w = jnp.exp(e)
                msg_buf[k, pl.ds(D, 16)] = w
                for g in range(G):
                    ha = (16 * g) // HD
                    hb = (16 * g + 15) // HD
                    if ha == hb:
                        wb = w[ha]
                    else:
                        wb = jnp.where(lane < hb * HD - 16 * g, w[ha], w[hb])
                    msg_buf[k, pl.ds(16 * g, 16)] = (
                        fb[k, pl.ds(16 * g, 16)] * wb)
                return carry2

            lax.fori_loop(0, CK, edge_body, 0)
            pltpu.sync_copy(msg_buf, accum.at[db], add=True)

        if dbuf:
            prefetch(0, 0)

            def pair_body(i, carry):
                prefetch(2 * i + 1, 1)
                process(0)

                @pl.when(2 * i + 2 < NCH)
                def _():
                    prefetch(2 * i + 2, 0)

                process(1)
                return carry

            lax.fori_loop(0, NCH // 2, pair_body, 0)
            if NCH % 2 == 1:
                process(0)
        else:
            def chunk_body(ci, carry):
                prefetch(ci, 0)
                process(0)
                return carry

            lax.fori_loop(0, NCH, chunk_body, 0)
        plsc.subcore_barrier()
        pltpu.sync_copy(accum.at[pl.ds(r0, ROWS_PER)],
                        out.at[c, pl.ds(r0, ROWS_PER)])

    return edge_kernel


_EDGE_K = {128: _make_edge_kernel(128, 80, True),
           80: _make_edge_kernel(80, 80, True, nheads=2)}


def _mm_body(x_ref, w_ref, o_ref):
    o_ref[...] = jnp.dot(x_ref[...], w_ref[...],
                         preferred_element_type=jnp.float32)


def _matmul(x, w):
    k = w.shape[1]
    return pl.pallas_call(
        _mm_body,
        grid=(N // BN,),
        in_specs=[pl.BlockSpec((BN, x.shape[1]), lambda i: (i, 0)),
                  pl.BlockSpec((x.shape[1], k), lambda i: (0, 0))],
        out_specs=pl.BlockSpec((BN, k), lambda i: (i, 0)),
        out_shape=jax.ShapeDtypeStruct((N, k), jnp.float32),
    )(x, w)


def _post01_body(p0_ref, p1_ref, sk_ref, sb_ref, b_ref, o_ref):
    lane = lax.broadcasted_iota(jnp.int32, (BN, 256), 1)
    p = jnp.where(lane < 132, p0_ref[...] + p1_ref[...], 0.0)
    denb = jnp.dot(p, b_ref[...], preferred_element_type=jnp.float32) + 1e-9
    msg = jnp.where(lane < 128, p / denb, 0.0)
    r = msg + sk_ref[...] + sb_ref[...]
    o_ref[...] = jnp.where(r > 0.0, r, jnp.exp(r) - 1.0)


def _post2_body(p0_ref, p1_ref, sk_ref, sb_ref, b_ref, s_ref, o_ref):
    lane = lax.broadcasted_iota(jnp.int32, (BN, 256), 1)
    p = jnp.where(lane < 164, p0_ref[...] + p1_ref[...], 0.0)
    denb = jnp.dot(p, b_ref[...], preferred_element_type=jnp.float32) + 1e-9
    msg = jnp.where(lane < 160, p / denb, 0.0)
    ms = jnp.dot(msg, s_ref[...], preferred_element_type=jnp.float32)
    m = 0.25 * ms + sk_ref[...] + sb_ref[...]
    mm = jnp.where(lane < 40, m, -1e30)
    mx = jnp.max(mm, axis=1, keepdims=True)
    ex = jnp.where(lane < 40, jnp.exp(mm - mx), 0.0)
    lse = jnp.log(jnp.sum(ex, axis=1, keepdims=True))
    o_ref[...] = mm - mx - lse


def _full_spec(shape):
    return pl.BlockSpec(shape, lambda i: tuple(0 for _ in shape))


def _post_call(body, args, extra_consts):
    specs = [pl.BlockSpec((BN, 256), lambda i: (i, 0)),
             pl.BlockSpec((BN, 256), lambda i: (i, 0)),
             pl.BlockSpec((BN, 256), lambda i: (i, 0)),
             _full_spec((1, 256))]
    specs += [_full_spec((256, 256)) for _ in extra_consts]
    return pl.pallas_call(
        body,
        grid=(N // BN,),
        in_specs=specs,
        out_specs=pl.BlockSpec((BN, 256), lambda i: (i, 0)),
        out_shape=jax.ShapeDtypeStruct((N, 256), jnp.float32),
    )(*args, *extra_consts)


def _np_bmat(d, hd):
    b = np.zeros((256, 256), np.float32)
    for h in range(HEADS):
        b[d + h, hd * h:hd * h + hd] = 1.0
    return jnp.asarray(b)


def _np_smat(hd):
    s = np.zeros((256, 256), np.float32)
    for h in range(HEADS):
        for j in range(hd):
            s[hd * h + j, j] = 1.0
    return jnp.asarray(s)


def _pad_cols(a, w):
    return jnp.pad(a, ((0, 0), (0, w - a.shape[1])))


def _layer_pre(h, W, al, ar, sW, D):
    hd = D // HEADS
    w3 = W.reshape(W.shape[0], HEADS, hd)
    wal = jnp.sum(w3 * al[None], axis=-1)
    war = jnp.sum(w3 * ar[None], axis=-1)
    wcat = jnp.concatenate([W, wal, war, sW], axis=1)
    kpad = -(-wcat.shape[1] // 128) * 128
    P = _matmul(h, _pad_cols(wcat, kpad))
    featx = _pad_cols(P[:, :D + 4], D + 16)
    qd = _pad_cols(P[:, D + 4:D + 8], 16)
    skip = P[:, D + 8:D + 8 + sW.shape[1]]
    return featx, qd, skip


def kernel(x, edge_index, W0, al0, ar0, sW0, sb0, W1, al1, ar1, sW1, sb1,
           W2, al2, ar2, sW2, sb2):
    src = edge_index[0]
    dst = edge_index[1]
    zeros01 = jnp.zeros((NPAD, 144), jnp.float32)
    zeros80 = jnp.zeros((NPAD, 96), jnp.float32)
    D2 = 160
    b01 = _np_bmat(128, 32)
    b2 = _np_bmat(160, 40)
    s2 = _np_smat(40)

    featx, qd, skip = _layer_pre(x, W0, al0, ar0, sW0, 128)
    parts = _EDGE_K[128](featx, qd, src, dst, zeros01)[:, :N]
    h = _post_call(_post01_body,
                   (_pad_cols(parts[0], 256), _pad_cols(parts[1], 256),
                    _pad_cols(skip, 256),
                    _pad_cols((sb0)[None, :], 256)),
                   (b01,))[:, :128]

    featx, qd, skip = _layer_pre(h, W1, al1, ar1, sW1, 128)
    parts = _EDGE_K[128](featx, qd, src, dst, zeros01)[:, :N]
    h = _post_call(_post01_body,
                   (_pad_cols(parts[0], 256), _pad_cols(parts[1], 256),
                    _pad_cols(skip, 256),
                    _pad_cols((sb1)[None, :], 256)),
                   (b01,))[:, :128]

    hd2 = D2 // HEADS
    w3 = W2.reshape(W2.shape[0], HEADS, hd2)
    wal = jnp.sum(w3 * al2[None], axis=-1)
    war = jnp.sum(w3 * ar2[None], axis=-1)
    wcat = jnp.concatenate([W2, wal, war, sW2], axis=1)
    P = _matmul(h, _pad_cols(wcat, 256))
    fxA = _pad_cols(jnp.concatenate([P[:, 0:80], P[:, 160:162]], axis=1), 96)
    fxB = _pad_cols(jnp.concatenate([P[:, 80:160], P[:, 162:164]], axis=1), 96)
    qdA = _pad_cols(P[:, 164:166], 16)
    qdB = _pad_cols(P[:, 166:168], 16)
    skip = P[:, 168:208]
    pA = _EDGE_K[80](fxA, qdA, src, dst, zeros80)[:, :N]
    pB = _EDGE_K[80](fxB, qdB, src, dst, zeros80)[:, :N]
    p0 = jnp.concatenate([pA[0, :, 0:80], pB[0, :, 0:80],
                          pA[0, :, 80:82], pB[0, :, 80:82]], axis=1)
    p1 = jnp.concatenate([pA[1, :, 0:80], pB[1, :, 0:80],
                          pA[1, :, 80:82], pB[1, :, 80:82]], axis=1)
    out = _post_call(_post2_body,
                     (_pad_cols(p0, 256), _pad_cols(p1, 256),
                      _pad_cols(skip, 256),
                      _pad_cols((sb2)[None, :], 256)),
                     (b2, s2))[:, :40]
    return out
